# Initial kernel scaffold; baseline (speedup 1.0000x reference)
import jax
import jax.numpy as jnp
from jax.experimental import pallas as pl


def kernel(x, edge_index):
    raise NotImplementedError("write your pallas kernel here")


# speedup vs baseline: 18.7891x; 18.7891x over previous
import functools
import math

import jax
import jax.numpy as jnp
from jax import lax
from jax.experimental import pallas as pl
from jax.experimental.pallas import tpu as pltpu
from jax.experimental.pallas import tpu_sc as plsc

N_NODES = 10000
N_EDGES = 160000
D_FEAT = 256
PRIV = 64
HALF = 128
CHUNK = 128
N_CHUNKS = N_EDGES // CHUNK
N_CHUNKS_PAD = 1256
NS = 16
NC = 2
NW = NC * NS
ROW_WIN = 632
N_PAD = 10240
U_SL = N_PAD // NS
HIST_N = 20480
HIST_SL = HIST_N // NS
MAXC_H = 48
MAXC_P = 88

_E = math.exp(1.0)
A_COEF = (_E + 1.0) / (_E - 1.0)
B_COEF = 0.1 - 1.0 / (_E - 1.0)

BLK = 400
NBLK = N_NODES // BLK

_sc_mesh = plsc.VectorSubcoreMesh(core_axis_name="c", subcore_axis_name="s")


@functools.partial(
    pl.kernel,
    out_type=jax.ShapeDtypeStruct((NC * HIST_N,), jnp.float32),
    mesh=_sc_mesh,
    scratch_types=[
        pltpu.VMEM((MAXC_H, CHUNK), jnp.int32),
        pltpu.VMEM((MAXC_H, CHUNK), jnp.int32),
        pltpu.VMEM((1, CHUNK), jnp.int32),
        pltpu.VMEM((1, CHUNK), jnp.int32),
        pltpu.VMEM((1, CHUNK), jnp.float32),
        pltpu.VMEM((1, CHUNK), jnp.float32),
        pltpu.VMEM((1, HIST_SL), jnp.float32),
        pltpu.VMEM_SHARED((HIST_N,), jnp.float32),
    ],
)
def _hist_kernel(rows_hbm, cols_hbm, out_hbm, rbuf, cbuf, rstage, rpbuf, sbuf,
                 ones, zbuf, acc):
    h = lax.axis_index("c")
    s = lax.axis_index("s")
    w = h * NS + s
    c_lo = (w * N_CHUNKS) // NW
    c_hi = ((w + 1) * N_CHUNKS) // NW
    start = pl.multiple_of((c_lo // 8) * 8, 8)

    zeros16 = jnp.zeros((16,), jnp.float32)
    for k in range(HIST_SL // 16):
        zbuf[0, pl.ds(k * 16, 16)] = zeros16
    for k in range(CHUNK // 16):
        ones[0, pl.ds(k * 16, 16)] = zeros16 + 1.0
    pltpu.sync_copy(zbuf.at[0], acc.at[pl.ds(s * HIST_SL, HIST_SL)])
    pltpu.sync_copy(rows_hbm.at[pl.ds(start, MAXC_H)], rbuf)
    pltpu.sync_copy(cols_hbm.at[pl.ds(start, MAXC_H)], cbuf)
    plsc.subcore_barrier()

    def chunk_body(j, carry):
        for k in range(CHUNK // 16):
            sl = pl.ds(k * 16, 16)
            rv = rbuf[j, sl]
            cv = cbuf[j, sl]
            rstage[0, sl] = rv
            rpbuf[0, sl] = rv + N_NODES
            sbuf[0, sl] = jnp.where(rv == cv, 1.0, 0.0)
        pltpu.sync_copy(ones.at[0], acc.at[rstage.at[0]], add=True)
        pltpu.sync_copy(sbuf.at[0], acc.at[rpbuf.at[0]], add=True)
        return carry

    lax.fori_loop(c_lo - start, c_hi - start, chunk_body, 0)
    plsc.subcore_barrier()
    pltpu.sync_copy(acc.at[pl.ds(s * HIST_SL, HIST_SL)],
                    out_hbm.at[pl.ds(h * HIST_N + s * HIST_SL, HIST_SL)])


def _prep_body(x_ref, degp_ref, selfp_ref, y_ref, z_ref, dinv_ref, lwdinv_ref):
    degraw = jnp.sum(degp_ref[...], axis=1, keepdims=True)
    selfsum = jnp.sum(selfp_ref[...], axis=1, keepdims=True)
    lw = jnp.where(selfsum == 0.0, 1.0, 0.0)
    dinv = lax.rsqrt(degraw + lw)
    yb = x_ref[...] * dinv
    y_ref[...] = yb[None]
    z_ref[...] = (lw * yb)[None]
    dinv_ref[...] = dinv
    lwdinv_ref[...] = lw * dinv


_prep = pl.pallas_call(
    _prep_body,
    grid=(NC, NBLK),
    in_specs=[
        pl.BlockSpec((BLK, HALF), lambda h, i: (i, h)),
        pl.BlockSpec((BLK, 2), lambda h, i: (i, 0)),
        pl.BlockSpec((BLK, 2), lambda h, i: (NBLK + i, 0)),
    ],
    out_specs=[
        pl.BlockSpec((1, BLK, HALF), lambda h, i: (h, i, 0)),
        pl.BlockSpec((1, BLK, HALF), lambda h, i: (h, i, 0)),
        pl.BlockSpec((BLK, 1), lambda h, i: (i, 0)),
        pl.BlockSpec((BLK, 1), lambda h, i: (i, 0)),
    ],
    out_shape=[
        jax.ShapeDtypeStruct((NC, N_NODES, HALF), jnp.float32),
        jax.ShapeDtypeStruct((NC, N_NODES, HALF), jnp.float32),
        jax.ShapeDtypeStruct((N_NODES, 1), jnp.float32),
        jax.ShapeDtypeStruct((N_NODES, 1), jnp.float32),
    ],
)


@functools.partial(
    pl.kernel,
    out_type=[
        jax.ShapeDtypeStruct((NC * N_NODES, HALF), jnp.float32),
        jax.ShapeDtypeStruct((NC * N_PAD,), jnp.float32),
    ],
    mesh=_sc_mesh,
    scratch_types=[
        pltpu.VMEM((MAXC_P, CHUNK), jnp.int32),
        pltpu.VMEM((MAXC_P, CHUNK), jnp.int32),
        pltpu.VMEM((CHUNK, HALF), jnp.float32),
        pltpu.VMEM((CHUNK,), jnp.float32),
        pltpu.VMEM((1, CHUNK), jnp.int32),
        pltpu.VMEM((1, CHUNK), jnp.int32),
        pltpu.VMEM_SHARED((N_NODES, HALF), jnp.float32),
        pltpu.VMEM_SHARED((N_PAD,), jnp.float32),
        pltpu.SemaphoreType.DMA,
        pltpu.SemaphoreType.DMA,
    ],
)
def _prop_kernel(rows_hbm, cols_hbm, y_hbm, z_hbm, dinv2_hbm, lwdinv_hbm,
                 out_hbm, outu_hbm, rbuf, cbuf, gbuf, dbuf, rstage, cstage,
                 acc, uacc, sem, sem2):
    h = lax.axis_index("c")
    s = lax.axis_index("s")
    c_lo = (s * N_CHUNKS) // NS
    c_hi = ((s + 1) * N_CHUNKS) // NS
    start = pl.multiple_of((c_lo // 8) * 8, 8)
    r_lo = pl.multiple_of(((s * N_NODES // NS) // 8) * 8, 8)

    pltpu.sync_copy(z_hbm.at[pl.ds(h * N_NODES + r_lo, ROW_WIN)],
                    acc.at[pl.ds(r_lo, ROW_WIN)])
    pltpu.sync_copy(lwdinv_hbm.at[pl.ds(s * U_SL, U_SL)],
                    uacc.at[pl.ds(s * U_SL, U_SL)])
    pltpu.sync_copy(rows_hbm.at[pl.ds(start, MAXC_P)], rbuf)
    pltpu.sync_copy(cols_hbm.at[pl.ds(start, MAXC_P)], cbuf)

    off = h * N_NODES
    plsc.subcore_barrier()

    def edge_body(j, carry):
        for k in range(CHUNK // 16):
            sl = pl.ds(k * 16, 16)
            rstage[0, sl] = rbuf[j, sl] + off
            cstage[0, sl] = cbuf[j, sl]
        cy = pltpu.async_copy(y_hbm.at[rstage.at[0]], gbuf, sem)
        cd = pltpu.async_copy(dinv2_hbm.at[rstage.at[0]], dbuf, sem2)
        cy.wait()
        cd.wait()
        pltpu.sync_copy(gbuf, acc.at[cstage.at[0]], add=True)
        pltpu.sync_copy(dbuf, uacc.at[cstage.at[0]], add=True)
        return carry

    lax.fori_loop(c_lo - start, c_hi - start, edge_body, 0)
    plsc.subcore_barrier()
    pltpu.sync_copy(acc.at[pl.ds(r_lo, ROW_WIN)],
                    out_hbm.at[pl.ds(h * N_NODES + r_lo, ROW_WIN)])
    pltpu.sync_copy(uacc.at[pl.ds(s * U_SL, U_SL)],
                    outu_hbm.at[pl.ds(h * N_PAD + s * U_SL, U_SL)])


def _fin_body(t0_ref, t1_ref, u_ref, dinv_ref, o_ref):
    t0 = t0_ref[0]
    t1 = t1_ref[0]
    dv = dinv_ref[...]
    u = u_ref[...]
    lane = lax.broadcasted_iota(jnp.int32, (BLK, HALF), 1)
    o0 = jnp.where(lane < PRIV, A_COEF * t0 + B_COEF * u, t0) * dv
    o1 = t1 * dv
    o_ref[...] = jnp.concatenate([o0, o1], axis=1)


_fin = pl.pallas_call(
    _fin_body,
    grid=(NBLK,),
    in_specs=[
        pl.BlockSpec((1, BLK, HALF), lambda i: (0, i, 0)),
        pl.BlockSpec((1, BLK, HALF), lambda i: (1, i, 0)),
        pl.BlockSpec((BLK, 1), lambda i: (i, 0)),
        pl.BlockSpec((BLK, 1), lambda i: (i, 0)),
    ],
    out_specs=pl.BlockSpec((BLK, D_FEAT), lambda i: (i, 0)),
    out_shape=jax.ShapeDtypeStruct((N_NODES, D_FEAT), jnp.float32),
)


def kernel(x, edge_index):
    rows = jnp.pad(edge_index[0].reshape(N_CHUNKS, CHUNK),
                   ((0, N_CHUNKS_PAD - N_CHUNKS), (0, 0)))
    cols = jnp.pad(edge_index[1].reshape(N_CHUNKS, CHUNK),
                   ((0, N_CHUNKS_PAD - N_CHUNKS), (0, 0)))
    hist = _hist_kernel(rows, cols)
    hist_t = hist.reshape(NC, HIST_N).T
    y3, z3, dinv, lwdinv = _prep(x, hist_t, hist_t)
    y = y3.reshape(NC * N_NODES, HALF)
    z = z3.reshape(NC * N_NODES, HALF)
    dinv1 = dinv.reshape(-1)
    dinv2 = jnp.concatenate([dinv1, dinv1])
    lwdinv_p = jnp.pad(lwdinv.reshape(-1), (0, N_PAD - N_NODES))
    t, u2 = _prop_kernel(rows, cols, y, z, dinv2, lwdinv_p)
    t3 = t.reshape(NC, N_NODES, HALF)
    u = u2[:N_NODES].reshape(N_NODES, 1)
    return _fin(t3, t3, u, dinv)


# speedup vs baseline: 23.4028x; 1.2456x over previous
import functools
import math

import jax
import jax.numpy as jnp
from jax import lax
from jax.experimental import pallas as pl
from jax.experimental.pallas import tpu as pltpu
from jax.experimental.pallas import tpu_sc as plsc

N_NODES = 10000
N_EDGES = 160000
D_FEAT = 256
PRIV = 64
HALF = 128
CHUNK = 128
N_CHUNKS = N_EDGES // CHUNK
N_CHUNKS_PAD = 1256
NS = 16
NC = 2
NW = NC * NS
ROW_WIN = 632
N_PAD = 10240
U_SL = N_PAD // NS
HIST_N = 20480
HIST_SL = HIST_N // NS
MAXC_H = 48
MAXC_P = 88

_E = math.exp(1.0)
A_COEF = (_E + 1.0) / (_E - 1.0)
B_COEF = 0.1 - 1.0 / (_E - 1.0)

BLK = 400
NBLK = N_NODES // BLK

_sc_mesh = plsc.VectorSubcoreMesh(core_axis_name="c", subcore_axis_name="s")


@functools.partial(
    pl.kernel,
    out_type=jax.ShapeDtypeStruct((NC * HIST_N,), jnp.float32),
    mesh=_sc_mesh,
    scratch_types=[
        pltpu.VMEM((MAXC_H, CHUNK), jnp.int32),
        pltpu.VMEM((MAXC_H, CHUNK), jnp.int32),
        pltpu.VMEM((1, CHUNK), jnp.int32),
        pltpu.VMEM((1, CHUNK), jnp.int32),
        pltpu.VMEM((1, CHUNK), jnp.float32),
        pltpu.VMEM((1, CHUNK), jnp.float32),
        pltpu.VMEM((1, HIST_SL), jnp.float32),
        pltpu.VMEM_SHARED((HIST_N,), jnp.float32),
    ],
)
def _hist_kernel(rows_hbm, cols_hbm, out_hbm, rbuf, cbuf, rstage, rpbuf, sbuf,
                 ones, zbuf, acc):
    h = lax.axis_index("c")
    s = lax.axis_index("s")
    w = h * NS + s
    c_lo = (w * N_CHUNKS) // NW
    c_hi = ((w + 1) * N_CHUNKS) // NW
    start = pl.multiple_of((c_lo // 8) * 8, 8)

    zeros16 = jnp.zeros((16,), jnp.float32)
    for k in range(HIST_SL // 16):
        zbuf[0, pl.ds(k * 16, 16)] = zeros16
    for k in range(CHUNK // 16):
        ones[0, pl.ds(k * 16, 16)] = zeros16 + 1.0
    pltpu.sync_copy(zbuf.at[0], acc.at[pl.ds(s * HIST_SL, HIST_SL)])
    pltpu.sync_copy(rows_hbm.at[pl.ds(start, MAXC_H)], rbuf)
    pltpu.sync_copy(cols_hbm.at[pl.ds(start, MAXC_H)], cbuf)
    plsc.subcore_barrier()

    def chunk_body(j, carry):
        for k in range(CHUNK // 16):
            sl = pl.ds(k * 16, 16)
            rv = rbuf[j, sl]
            cv = cbuf[j, sl]
            rstage[0, sl] = rv
            rpbuf[0, sl] = rv + N_NODES
            sbuf[0, sl] = jnp.where(rv == cv, 1.0, 0.0)
        pltpu.sync_copy(ones.at[0], acc.at[rstage.at[0]], add=True)
        pltpu.sync_copy(sbuf.at[0], acc.at[rpbuf.at[0]], add=True)
        return carry

    lax.fori_loop(c_lo - start, c_hi - start, chunk_body, 0)
    plsc.subcore_barrier()
    pltpu.sync_copy(acc.at[pl.ds(s * HIST_SL, HIST_SL)],
                    out_hbm.at[pl.ds(h * HIST_N + s * HIST_SL, HIST_SL)])


def _prep_body(x_ref, degp_ref, selfp_ref, y_ref, z_ref, dinv_ref, lwdinv_ref):
    degraw = jnp.sum(degp_ref[...], axis=1, keepdims=True)
    selfsum = jnp.sum(selfp_ref[...], axis=1, keepdims=True)
    lw = jnp.where(selfsum == 0.0, 1.0, 0.0)
    dinv = lax.rsqrt(degraw + lw)
    yb = x_ref[...] * dinv
    y_ref[...] = yb[None]
    z_ref[...] = (lw * yb)[None]
    dinv_ref[...] = dinv
    lwdinv_ref[...] = lw * dinv


_prep = pl.pallas_call(
    _prep_body,
    grid=(NC, NBLK),
    in_specs=[
        pl.BlockSpec((BLK, HALF), lambda h, i: (i, h)),
        pl.BlockSpec((BLK, 2), lambda h, i: (i, 0)),
        pl.BlockSpec((BLK, 2), lambda h, i: (NBLK + i, 0)),
    ],
    out_specs=[
        pl.BlockSpec((1, BLK, HALF), lambda h, i: (h, i, 0)),
        pl.BlockSpec((1, BLK, HALF), lambda h, i: (h, i, 0)),
        pl.BlockSpec((BLK, 1), lambda h, i: (i, 0)),
        pl.BlockSpec((BLK, 1), lambda h, i: (i, 0)),
    ],
    out_shape=[
        jax.ShapeDtypeStruct((NC, N_NODES, HALF), jnp.float32),
        jax.ShapeDtypeStruct((NC, N_NODES, HALF), jnp.float32),
        jax.ShapeDtypeStruct((N_NODES, 1), jnp.float32),
        jax.ShapeDtypeStruct((N_NODES, 1), jnp.float32),
    ],
)


N_GRP = N_CHUNKS_PAD // 8


@functools.partial(
    pl.kernel,
    out_type=[
        jax.ShapeDtypeStruct((NC * N_NODES, HALF), jnp.float32),
        jax.ShapeDtypeStruct((NC * N_PAD,), jnp.float32),
    ],
    mesh=_sc_mesh,
    scratch_types=[
        pltpu.VMEM((8, CHUNK), jnp.int32),
        pltpu.VMEM((8, CHUNK), jnp.int32),
        pltpu.VMEM((8, CHUNK), jnp.int32),
        pltpu.VMEM((8, CHUNK), jnp.int32),
        pltpu.VMEM((CHUNK, HALF), jnp.float32),
        pltpu.VMEM((CHUNK, HALF), jnp.float32),
        pltpu.VMEM((CHUNK,), jnp.float32),
        pltpu.VMEM((CHUNK,), jnp.float32),
        pltpu.VMEM((1, CHUNK), jnp.int32),
        pltpu.VMEM((1, CHUNK), jnp.int32),
        pltpu.VMEM((1, CHUNK), jnp.int32),
        pltpu.VMEM((1, CHUNK), jnp.int32),
        pltpu.VMEM_SHARED((N_NODES, HALF), jnp.float32),
        pltpu.VMEM_SHARED((N_PAD,), jnp.float32),
        pltpu.SemaphoreType.DMA,
        pltpu.SemaphoreType.DMA,
        pltpu.SemaphoreType.DMA,
        pltpu.SemaphoreType.DMA,
        pltpu.SemaphoreType.DMA,
    ],
)
def _prop_kernel(rows_hbm, cols_hbm, y_hbm, z_hbm, dinv2_hbm, lwdinv_hbm,
                 out_hbm, outu_hbm, rbufA, rbufB, cbufA, cbufB, gbuf0, gbuf1,
                 dbuf0, dbuf1, rstage0, rstage1, cstage0, cstage1, acc, uacc,
                 semg0, semg1, semd0, semd1, semi):
    h = lax.axis_index("c")
    s = lax.axis_index("s")
    g_lo = (s * N_GRP) // NS
    g_hi = ((s + 1) * N_GRP) // NS
    r_lo = pl.multiple_of(((s * N_NODES // NS) // 8) * 8, 8)

    pltpu.sync_copy(z_hbm.at[pl.ds(h * N_NODES + r_lo, ROW_WIN)],
                    acc.at[pl.ds(r_lo, ROW_WIN)])
    pltpu.sync_copy(lwdinv_hbm.at[pl.ds(s * U_SL, U_SL)],
                    uacc.at[pl.ds(s * U_SL, U_SL)])
    pltpu.sync_copy(rows_hbm.at[g_lo], rbufA)
    pltpu.sync_copy(cols_hbm.at[g_lo], cbufA)
    off = h * N_NODES
    plsc.subcore_barrier()

    slots = ((rstage0, cstage0, gbuf0, dbuf0, semg0, semd0),
             (rstage1, cstage1, gbuf1, dbuf1, semg1, semd1))

    def stage_fire(rb, cb, i, q):
        rstage, cstage, gbuf, dbuf, semg, semd = slots[i % 2]
        for k in range(CHUNK // 16):
            sl = pl.ds(k * 16, 16)
            rstage[0, sl] = rb[i, sl] + off
            cstage[0, sl] = cb[i, sl]
        pltpu.async_copy(y_hbm.at[rstage.at[0]], gbuf, semg)
        pltpu.async_copy(dinv2_hbm.at[rstage.at[0]], dbuf, semd)

    def drain_scatter(i):
        rstage, cstage, gbuf, dbuf, semg, semd = slots[i % 2]
        pltpu.make_async_copy(y_hbm.at[rstage.at[0]], gbuf, semg).wait()
        pltpu.make_async_copy(dinv2_hbm.at[rstage.at[0]], dbuf, semd).wait()
        pltpu.sync_copy(gbuf, acc.at[cstage.at[0]], add=True)
        pltpu.sync_copy(dbuf, uacc.at[cstage.at[0]], add=True)

    def process_group(rb, cb, q):
        for i in range(8):
            c = q * 8 + i

            @pl.when(c < N_CHUNKS)
            def _():
                stage_fire(rb, cb, i, q)

            if i > 0:
                @pl.when(c - 1 < N_CHUNKS)
                def _():
                    drain_scatter(i - 1)

        @pl.when(q * 8 + 7 < N_CHUNKS)
        def _():
            drain_scatter(7)

    def pair_body(m, carry):
        qa = g_lo + 2 * m
        qb = qa + 1

        @pl.when(qb < g_hi)
        def _():
            pltpu.async_copy(rows_hbm.at[qb], rbufB, semi)
            pltpu.async_copy(cols_hbm.at[qb], cbufB, semi)

        process_group(rbufA, cbufA, qa)

        @pl.when(qb < g_hi)
        def _():
            pltpu.make_async_copy(rows_hbm.at[qb], rbufB, semi).wait()
            pltpu.make_async_copy(cols_hbm.at[qb], cbufB, semi).wait()

            @pl.when(qb + 1 < g_hi)
            def _():
                pltpu.async_copy(rows_hbm.at[qb + 1], rbufA, semi)
                pltpu.async_copy(cols_hbm.at[qb + 1], cbufA, semi)

            process_group(rbufB, cbufB, qb)

            @pl.when(qb + 1 < g_hi)
            def _():
                pltpu.make_async_copy(rows_hbm.at[qb + 1], rbufA, semi).wait()
                pltpu.make_async_copy(cols_hbm.at[qb + 1], cbufA, semi).wait()

        return carry

    lax.fori_loop(0, (g_hi - g_lo + 1) // 2, pair_body, 0)
    plsc.subcore_barrier()
    pltpu.sync_copy(acc.at[pl.ds(r_lo, ROW_WIN)],
                    out_hbm.at[pl.ds(h * N_NODES + r_lo, ROW_WIN)])
    pltpu.sync_copy(uacc.at[pl.ds(s * U_SL, U_SL)],
                    outu_hbm.at[pl.ds(h * N_PAD + s * U_SL, U_SL)])


def _fin_body(t0_ref, t1_ref, u_ref, dinv_ref, o_ref):
    t0 = t0_ref[0]
    t1 = t1_ref[0]
    dv = dinv_ref[...]
    u = u_ref[...]
    lane = lax.broadcasted_iota(jnp.int32, (BLK, HALF), 1)
    o0 = jnp.where(lane < PRIV, A_COEF * t0 + B_COEF * u, t0) * dv
    o1 = t1 * dv
    o_ref[...] = jnp.concatenate([o0, o1], axis=1)


_fin = pl.pallas_call(
    _fin_body,
    grid=(NBLK,),
    in_specs=[
        pl.BlockSpec((1, BLK, HALF), lambda i: (0, i, 0)),
        pl.BlockSpec((1, BLK, HALF), lambda i: (1, i, 0)),
        pl.BlockSpec((BLK, 1), lambda i: (i, 0)),
        pl.BlockSpec((BLK, 1), lambda i: (i, 0)),
    ],
    out_specs=pl.BlockSpec((BLK, D_FEAT), lambda i: (i, 0)),
    out_shape=jax.ShapeDtypeStruct((N_NODES, D_FEAT), jnp.float32),
)


def kernel(x, edge_index):
    rows = jnp.pad(edge_index[0].reshape(N_CHUNKS, CHUNK),
                   ((0, N_CHUNKS_PAD - N_CHUNKS), (0, 0)))
    cols = jnp.pad(edge_index[1].reshape(N_CHUNKS, CHUNK),
                   ((0, N_CHUNKS_PAD - N_CHUNKS), (0, 0)))
    hist = _hist_kernel(rows, cols)
    hist_t = hist.reshape(NC, HIST_N).T
    y3, z3, dinv, lwdinv = _prep(x, hist_t, hist_t)
    y = y3.reshape(NC * N_NODES, HALF)
    z = z3.reshape(NC * N_NODES, HALF)
    dinv1 = dinv.reshape(-1)
    dinv2 = jnp.concatenate([dinv1, dinv1])
    lwdinv_p = jnp.pad(lwdinv.reshape(-1), (0, N_PAD - N_NODES))
    rows3 = rows.reshape(N_CHUNKS_PAD // 8, 8, CHUNK)
    cols3 = cols.reshape(N_CHUNKS_PAD // 8, 8, CHUNK)
    t, u2 = _prop_kernel(rows3, cols3, y, z, dinv2, lwdinv_p)
    t3 = t.reshape(NC, N_NODES, HALF)
    u = u2[:N_NODES].reshape(N_NODES, 1)
    return _fin(t3, t3, u, dinv)
